# baseline (speedup 1.0000x reference)
import functools

import jax
import jax.numpy as jnp
from jax import lax
from jax.experimental import pallas as pl
from jax.experimental.pallas import tpu as pltpu
from jax.experimental.pallas import tpu_sc as plsc

_VOCAB = 151936
_D = 2048
_BATCH = 4
_SEQ = 8192

_NC = 2
_NS = 16
_NW = _NC * _NS

_B_TOTAL = _BATCH * _SEQ
_B_PER_W = _B_TOTAL // _NW
_NSLOT = 8
_LAG = 4
_C = 4
_NCHUNK = _B_PER_W // _C
_NP = _NCHUNK // _NSLOT


def _embed_kernel(idx_hbm, table_hbm, out_hbm, idx_v, bufs, gsem, ssem):
    wid = lax.axis_index("s") * _NC + lax.axis_index("c")
    base = wid * _B_PER_W

    pltpu.sync_copy(idx_hbm.at[wid], idx_v)

    def gather(j, slot):
        pltpu.async_copy(table_hbm.at[idx_v.at[j]], bufs.at[slot], gsem)

    def scatter(j, slot):
        pltpu.async_copy(bufs.at[slot], out_hbm.at[pl.ds(base + j * _C, _C)],
                         ssem)

    def wait_g(slot):
        pltpu.make_async_copy(table_hbm.at[pl.ds(0, _C)], bufs.at[slot],
                              gsem).wait()

    def wait_s(slot):
        pltpu.make_async_copy(bufs.at[slot], out_hbm.at[pl.ds(base, _C)],
                              ssem).wait()

    for b in range(_LAG):
        gather(b, b)
    for b in range(_LAG, _NSLOT):
        gather(b, b)
        wait_g(b - _LAG)
        scatter(b - _LAG, b - _LAG)

    def body(p, carry):
        j0 = p * _NSLOT
        for b in range(_NSLOT):
            j = j0 + b
            wait_s(b)
            gather(j, b)
            sl = (b + _LAG) % _NSLOT
            wait_g(sl)
            scatter(j - _LAG, sl)
        return carry

    lax.fori_loop(1, _NP, body, 0)

    jt = _NCHUNK - _LAG
    for b in range(_LAG):
        sl = (b + _LAG) % _NSLOT
        wait_g(sl)
        scatter(jt + b, sl)
    for b in range(_NSLOT):
        wait_s(b)


@jax.jit
def _embed(idx3, table):
    mesh = plsc.VectorSubcoreMesh(core_axis_name="c", subcore_axis_name="s")
    return pl.kernel(
        _embed_kernel,
        out_type=jax.ShapeDtypeStruct((_B_TOTAL, _D), jnp.float32),
        mesh=mesh,
        scratch_types=[
            pltpu.VMEM((_NCHUNK, _C), jnp.int32),
            pltpu.VMEM((_NSLOT, _C, _D), jnp.float32),
            pltpu.SemaphoreType.DMA,
            pltpu.SemaphoreType.DMA,
        ],
    )(idx3, table)


def kernel(input_ids, embed_table):
    idx3 = input_ids.reshape(_NW, _NCHUNK, _C).astype(jnp.int32)
    out = _embed(idx3, embed_table)
    return out.reshape(_BATCH, _SEQ, _D)


# speedup vs baseline: 1.0291x; 1.0291x over previous
import functools

import jax
import jax.numpy as jnp
from jax import lax
from jax.experimental import pallas as pl
from jax.experimental.pallas import tpu as pltpu
from jax.experimental.pallas import tpu_sc as plsc

_VOCAB = 151936
_D = 2048
_BATCH = 4
_SEQ = 8192

_NC = 2
_NS = 16
_NW = _NC * _NS

_B_TOTAL = _BATCH * _SEQ
_B_PER_W = _B_TOTAL // _NW
_NSLOT = 4
_LAG = 2
_C = 8
_NCHUNK = _B_PER_W // _C
_NP = _NCHUNK // _NSLOT


def _embed_kernel(idx_hbm, table_hbm, out_hbm, idx_v, bufs, gsem, ssem):
    wid = lax.axis_index("s") * _NC + lax.axis_index("c")
    base = wid * _B_PER_W

    pltpu.sync_copy(idx_hbm.at[wid], idx_v)

    def gather(j, slot):
        pltpu.async_copy(table_hbm.at[idx_v.at[j]], bufs.at[slot], gsem)

    def scatter(j, slot):
        pltpu.async_copy(bufs.at[slot], out_hbm.at[pl.ds(base + j * _C, _C)],
                         ssem)

    def wait_g(slot):
        pltpu.make_async_copy(table_hbm.at[pl.ds(0, _C)], bufs.at[slot],
                              gsem).wait()

    def wait_s(slot):
        pltpu.make_async_copy(bufs.at[slot], out_hbm.at[pl.ds(base, _C)],
                              ssem).wait()

    for b in range(_LAG):
        gather(b, b)
    for b in range(_LAG, _NSLOT):
        gather(b, b)
        wait_g(b - _LAG)
        scatter(b - _LAG, b - _LAG)

    def body(p, carry):
        j0 = p * _NSLOT
        for b in range(_NSLOT):
            j = j0 + b
            wait_s(b)
            gather(j, b)
            sl = (b + _LAG) % _NSLOT
            wait_g(sl)
            scatter(j - _LAG, sl)
        return carry

    lax.fori_loop(1, _NP, body, 0)

    jt = _NCHUNK - _LAG
    for b in range(_LAG):
        sl = (b + _LAG) % _NSLOT
        wait_g(sl)
        scatter(jt + b, sl)
    for b in range(_NSLOT):
        wait_s(b)


@jax.jit
def _embed(idx3, table):
    mesh = plsc.VectorSubcoreMesh(core_axis_name="c", subcore_axis_name="s")
    return pl.kernel(
        _embed_kernel,
        out_type=jax.ShapeDtypeStruct((_B_TOTAL, _D), jnp.float32),
        mesh=mesh,
        scratch_types=[
            pltpu.VMEM((_NCHUNK, _C), jnp.int32),
            pltpu.VMEM((_NSLOT, _C, _D), jnp.float32),
            pltpu.SemaphoreType.DMA,
            pltpu.SemaphoreType.DMA,
        ],
    )(idx3, table)


def kernel(input_ids, embed_table):
    idx3 = input_ids.reshape(_NW, _NCHUNK, _C).astype(jnp.int32)
    out = _embed(idx3, embed_table)
    return out.reshape(_BATCH, _SEQ, _D)
